# Initial kernel scaffold; baseline (speedup 1.0000x reference)
import jax
import jax.numpy as jnp
from jax.experimental import pallas as pl


def kernel(x, token_table, pos_table):
    raise NotImplementedError("write your pallas kernel here")


# speedup vs baseline: 1.1375x; 1.1375x over previous
import functools

import jax
import jax.numpy as jnp
from jax import lax
from jax.experimental import pallas as pl
from jax.experimental.pallas import tpu as pltpu
from jax.experimental.pallas import tpu_sc as plsc


def _make_sc_kernel(B, S, D, GATHER, CSEQ):
    NC, NS = 2, 16
    NW = NC * NS
    ROWS = B * S
    RPW = ROWS // NW
    SEQ_PER_W = RPW // S
    NCHUNK = SEQ_PER_W // CSEQ
    CROWS = CSEQ * S
    GPC = CROWS // GATHER
    GROWS_PER_W = RPW // GATHER

    assert RPW % S == 0 and SEQ_PER_W % CSEQ == 0 and CROWS % GATHER == 0
    assert GATHER <= 128 and NCHUNK % 2 == 0

    mesh = plsc.VectorSubcoreMesh(core_axis_name="c", subcore_axis_name="s")

    def body(x_hbm, tok_hbm, pos_hbm, out_hbm,
             idx_v, pos_v, rows0, rows1, gsem, osem):
        wid = lax.axis_index("s") * NC + lax.axis_index("c")
        grow0 = wid * GROWS_PER_W
        pltpu.sync_copy(x_hbm.at[pl.ds(grow0, GROWS_PER_W)], idx_v)
        pltpu.sync_copy(pos_hbm, pos_v)

        bufs = (rows0, rows1)

        def fire_gathers(c, buf):
            for g in range(GPC):
                pltpu.async_copy(tok_hbm.at[idx_v.at[c * GPC + g]],
                                 buf.at[g], gsem)

        def wait_gathers(c, buf):
            for g in range(GPC):
                pltpu.make_async_copy(tok_hbm.at[idx_v.at[c * GPC + g]],
                                      buf.at[g], gsem).wait()

        def out_slice(c):
            return out_hbm.at[pl.ds(grow0 + c * GPC, GPC)]

        def add_pos(buf):
            @pl.loop(0, GATHER)
            def _(i):
                for g in range(GPC):
                    for h in range(0, D, 16):
                        sl = (g, i, pl.ds(h, 16))
                        buf[sl] = buf[sl] + pos_v[sl]

        fire_gathers(0, bufs[0])

        @pl.loop(0, NCHUNK, step=2)
        def _(c0):
            for p in range(2):
                c = c0 + p
                cur, nxt = bufs[p], bufs[1 - p]
                wait_gathers(c, cur)
                @pl.when(c > 0)
                def _():
                    pltpu.make_async_copy(nxt, out_slice(c - 1), osem).wait()

                @pl.when(c < NCHUNK - 1)
                def _():
                    fire_gathers(c + 1, nxt)

                add_pos(cur)
                pltpu.async_copy(cur, out_slice(c), osem)

        pltpu.make_async_copy(bufs[(NCHUNK - 1) % 2],
                              out_slice(NCHUNK - 1), osem).wait()

    grid_rows = ROWS // GATHER
    return pl.kernel(
        body,
        out_type=jax.ShapeDtypeStruct((grid_rows, GATHER, D), jnp.float32),
        mesh=mesh,
        scratch_types=[
            pltpu.VMEM((GROWS_PER_W, GATHER), jnp.int32),
            pltpu.VMEM((GPC, GATHER, D), jnp.float32),
            pltpu.VMEM((GPC, GATHER, D), jnp.float32),
            pltpu.VMEM((GPC, GATHER, D), jnp.float32),
            pltpu.SemaphoreType.DMA,
            pltpu.SemaphoreType.DMA,
        ],
        compiler_params=pltpu.CompilerParams(use_tc_tiling_on_sc=False),
    )


@jax.jit
def kernel(x, token_table, pos_table):
    B, S = x.shape
    D = token_table.shape[1]
    GATHER = 100
    CSEQ = 4
    x2d = x.reshape(B * S // GATHER, GATHER).astype(jnp.int32)
    pos_pat = jnp.tile(pos_table, (CSEQ, 1)).reshape(
        CSEQ * S // GATHER, GATHER, D)
    sc = _make_sc_kernel(B, S, D, GATHER, CSEQ)
    out = sc(x2d, token_table, pos_pat)
    return out.reshape(B, S, D)


# speedup vs baseline: 1.4900x; 1.3099x over previous
import functools

import jax
import jax.numpy as jnp
from jax import lax
from jax.experimental import pallas as pl
from jax.experimental.pallas import tpu as pltpu
from jax.experimental.pallas import tpu_sc as plsc


def _make_sc_kernel(B, S, D, GATHER, CSEQ):
    NC, NS = 2, 16
    NW = NC * NS
    ROWS = B * S
    RPW = ROWS // NW
    SEQ_PER_W = RPW // S
    NCHUNK = SEQ_PER_W // CSEQ
    CROWS = CSEQ * S
    GPC = CROWS // GATHER
    GROWS_PER_W = RPW // GATHER
    GPS = S // GATHER

    assert RPW % S == 0 and SEQ_PER_W % CSEQ == 0 and S % GATHER == 0
    assert GATHER <= 128 and NCHUNK % 2 == 0

    mesh = plsc.VectorSubcoreMesh(core_axis_name="c", subcore_axis_name="s")

    def body(x_hbm, tok_hbm, pos_hbm, out_hbm,
             idx_v, pos_v, rows0, rows1, gsem, osem):
        wid = lax.axis_index("s") * NC + lax.axis_index("c")
        grow0 = wid * GROWS_PER_W
        seq0 = wid * SEQ_PER_W
        pltpu.sync_copy(x_hbm.at[pl.ds(grow0, GROWS_PER_W)], idx_v)
        pltpu.sync_copy(pos_hbm, pos_v)

        bufs = (rows0, rows1)

        def fire_gathers(c, buf):
            for q in range(CSEQ):
                for h in range(GPS):
                    pltpu.async_copy(
                        tok_hbm.at[idx_v.at[c * GPC + q * GPS + h]],
                        buf.at[q, pl.ds(h * GATHER, GATHER)], gsem)

        def wait_gathers(c, buf):
            for q in range(CSEQ):
                for h in range(GPS):
                    pltpu.make_async_copy(
                        tok_hbm.at[idx_v.at[c * GPC + q * GPS + h]],
                        buf.at[q, pl.ds(h * GATHER, GATHER)], gsem).wait()

        def out_slice(c):
            return out_hbm.at[pl.ds(seq0 + c * CSEQ, CSEQ)]

        def add_pos(buf):
            @pl.loop(0, S)
            def _(i):
                for h in range(0, D, 16):
                    p = pos_v[i, pl.ds(h, 16)]
                    for q in range(CSEQ):
                        sl = (q, i, pl.ds(h, 16))
                        buf[sl] = buf[sl] + p

        fire_gathers(0, bufs[0])

        @pl.loop(0, NCHUNK, step=2)
        def _(c0):
            for p in range(2):
                c = c0 + p
                cur, nxt = bufs[p], bufs[1 - p]
                wait_gathers(c, cur)
                @pl.when(c > 0)
                def _():
                    pltpu.make_async_copy(nxt, out_slice(c - 1), osem).wait()

                @pl.when(c < NCHUNK - 1)
                def _():
                    fire_gathers(c + 1, nxt)

                add_pos(cur)
                pltpu.async_copy(cur, out_slice(c), osem)

        pltpu.make_async_copy(bufs[(NCHUNK - 1) % 2],
                              out_slice(NCHUNK - 1), osem).wait()

    return pl.kernel(
        body,
        out_type=jax.ShapeDtypeStruct((B, S, D), jnp.float32),
        mesh=mesh,
        scratch_types=[
            pltpu.VMEM((GROWS_PER_W, GATHER), jnp.int32),
            pltpu.VMEM((S, D), jnp.float32),
            pltpu.VMEM((CSEQ, S, D), jnp.float32),
            pltpu.VMEM((CSEQ, S, D), jnp.float32),
            pltpu.SemaphoreType.DMA,
            pltpu.SemaphoreType.DMA,
        ],
        compiler_params=pltpu.CompilerParams(use_tc_tiling_on_sc=False),
    )


@jax.jit
def kernel(x, token_table, pos_table):
    B, S = x.shape
    D = token_table.shape[1]
    GATHER = 100
    CSEQ = 4
    x2d = x.reshape(B * S // GATHER, GATHER).astype(jnp.int32)
    sc = _make_sc_kernel(B, S, D, GATHER, CSEQ)
    return sc(x2d, token_table, pos_table)


# speedup vs baseline: 2.0286x; 1.3614x over previous
import functools

import jax
import jax.numpy as jnp
from jax import lax
from jax.experimental import pallas as pl
from jax.experimental.pallas import tpu as pltpu
from jax.experimental.pallas import tpu_sc as plsc


def _make_sc_kernel(B, S, D, GATHER, CSEQ):
    NC, NS = 2, 16
    NW = NC * NS
    ROWS = B * S
    RPW = ROWS // NW
    SEQ_PER_W = RPW // S
    NCHUNK = SEQ_PER_W // CSEQ
    CROWS = CSEQ * S
    GPC = CROWS // GATHER
    GROWS_PER_W = RPW // GATHER

    assert RPW % S == 0 and SEQ_PER_W % CSEQ == 0 and CROWS % GATHER == 0
    assert GATHER <= 128 and NCHUNK % 2 == 0

    mesh = plsc.VectorSubcoreMesh(core_axis_name="c", subcore_axis_name="s")

    def body(x_hbm, tok_hbm, pos_hbm, out_hbm,
             idx_v, pos_v, rows0, rows1, gsem, osem):
        tok = tok_hbm
        wid = lax.axis_index("s") * NC + lax.axis_index("c")
        grow0 = wid * GROWS_PER_W
        seq0 = wid * SEQ_PER_W
        pltpu.sync_copy(x_hbm.at[pl.ds(grow0, GROWS_PER_W)], idx_v)
        pltpu.sync_copy(pos_hbm, pos_v)

        bufs = (rows0, rows1)

        def fire_gathers(c, buf):
            for g in range(GPC):
                pltpu.async_copy(
                    tok.at[idx_v.at[c * GPC + g]],
                    buf.at[g // 2, pl.ds((g % 2) * GATHER, GATHER)], gsem)

        def wait_gathers(c, buf):
            for g in range(GPC):
                pltpu.make_async_copy(
                    tok.at[idx_v.at[c * GPC + g]],
                    buf.at[g // 2, pl.ds((g % 2) * GATHER, GATHER)],
                    gsem).wait()

        def out_slice(c):
            return out_hbm.at[pl.ds(seq0 + c * CSEQ, CSEQ), :, pl.ds(0, D)]

        def add_pos(buf):
            @pl.loop(0, S)
            def _(i):
                for h in range(0, D, 16):
                    p = pos_v[i, pl.ds(h, 16)]
                    for q in range(CSEQ):
                        sl = (q, i, pl.ds(h, 16))
                        buf[sl] = buf[sl] + p

        fire_gathers(0, bufs[0])

        @pl.loop(0, NCHUNK, step=2)
        def _(c0):
            for p in range(2):
                c = c0 + p
                cur, nxt = bufs[p], bufs[1 - p]
                wait_gathers(c, cur)
                @pl.when(c > 0)
                def _():
                    pltpu.make_async_copy(nxt, out_slice(c - 1), osem).wait()

                @pl.when(c < NCHUNK - 1)
                def _():
                    fire_gathers(c + 1, nxt)

                add_pos(cur)
                pltpu.async_copy(cur, out_slice(c), osem)

        pltpu.make_async_copy(bufs[(NCHUNK - 1) % 2],
                              out_slice(NCHUNK - 1), osem).wait()

    return pl.kernel(
        body,
        out_type=jax.ShapeDtypeStruct((B, S, 128), jnp.float32),
        mesh=mesh,
        scratch_types=[
            pltpu.VMEM((GROWS_PER_W, GATHER), jnp.int32),
            pltpu.VMEM((S, D), jnp.float32),
            pltpu.VMEM((CSEQ, S, D), jnp.float32),
            pltpu.VMEM((CSEQ, S, D), jnp.float32),
            pltpu.SemaphoreType.DMA,
            pltpu.SemaphoreType.DMA,
        ],
        compiler_params=pltpu.CompilerParams(use_tc_tiling_on_sc=False),
    )


@jax.jit
def kernel(x, token_table, pos_table):
    B, S = x.shape
    V, D = token_table.shape
    GATHER = 100
    CSEQ = 4
    x2d = x.reshape(B * S // GATHER, GATHER).astype(jnp.int32)
    tt128 = lax.optimization_barrier(token_table.reshape(V * D // 128, 128))
    tt = tt128.reshape(V, D)
    sc = _make_sc_kernel(B, S, D, GATHER, CSEQ)
    out = sc(x2d, tt, pos_table)
    return out[:, :, :D]
